# baseline (speedup 1.0000x reference)
import functools

import jax
import jax.numpy as jnp
from jax import lax
from jax.experimental import pallas as pl
from jax.experimental.pallas import tpu as pltpu
from jax.experimental.pallas import tpu_sc as plsc

N = 50000
NPAD = 50176
ROWS = 3136
RPS = 196
SL = 3136
E = 1600000
NW = 32
EPW = E // NW
CH = 2000
NCH = EPW // CH
VPC = CH // 16

_mesh = plsc.VectorSubcoreMesh(core_axis_name="c", subcore_axis_name="s")


@functools.partial(
    pl.kernel,
    out_type=jax.ShapeDtypeStruct((2, 16, RPS, 16), jnp.float32),
    mesh=_mesh,
    compiler_params=pltpu.CompilerParams(needs_layout_passes=False,
                                         use_tc_tiling_on_sc=False),
    scratch_types=[
        pltpu.VMEM((NPAD,), jnp.float32),
        pltpu.VMEM((ROWS, 16), jnp.float32),
        pltpu.VMEM((SL,), jnp.float32),
        pltpu.VMEM((SL,), jnp.float32),
        pltpu.VMEM((CH,), jnp.int32),
        pltpu.VMEM((CH,), jnp.int32),
        pltpu.VMEM((CH,), jnp.int32),
        pltpu.VMEM((CH,), jnp.int32),
        pltpu.VMEM((28, 112), jnp.int32),
        pltpu.VMEM((16,), jnp.float32),
        pltpu.VMEM((16,), jnp.float32),
        pltpu.VMEM((16,), jnp.float32),
        pltpu.VMEM_SHARED((NPAD,), jnp.float32),
        pltpu.VMEM_SHARED((ROWS, 16), jnp.float32),
        pltpu.SemaphoreType.DMA,
        pltpu.SemaphoreType.DMA,
        pltpu.SemaphoreType.DMA,
        pltpu.SemaphoreType.DMA,
        pltpu.SemaphoreType.DMA,
    ],
)
def _rgc_layer(pa, pb, ese, edst, wtab, cvec, bvec, ridx_in,
               out,
               x_local, acc, buf_a, buf_b, se0, se1, sd0, sd1,
               ridx, wt, cv, bv, x_shared, part,
               sem_a, sem_b, esem0, esem1, rsem):
    cid = lax.axis_index("c")
    sid = lax.axis_index("s")
    wid = cid * 16 + sid
    se = (se0, se1)
    sd = (sd0, sd1)
    esem = (esem0, esem1)

    pltpu.sync_copy(wtab, wt)
    pltpu.sync_copy(cvec, cv)
    pltpu.sync_copy(bvec, bv)

    cpa = pltpu.async_copy(pa.at[pl.ds(sid * SL, SL)], buf_a, sem_a)
    cpb = pltpu.async_copy(pb.at[pl.ds(sid * SL, SL)], buf_b, sem_b)
    cri = pltpu.async_copy(ridx_in, ridx, rsem)

    zeros = jnp.zeros((16,), jnp.float32)

    def _zero(r, carry):
        acc[r, :] = zeros
        return carry

    lax.fori_loop(0, ROWS, _zero, None)
    cpa.wait()
    cpb.wait()

    def _addp(i, carry):
        s16 = pl.ds(i * 16, 16)
        buf_a[s16] = buf_a[s16] + buf_b[s16]
        return carry

    lax.fori_loop(0, RPS, _addp, None)
    pltpu.sync_copy(buf_a, x_shared.at[pl.ds(sid * SL, SL)])
    plsc.subcore_barrier()

    pltpu.sync_copy(x_shared, x_local)

    ebase = wid * EPW

    def _start(ch, b):
        off = ebase + ch * CH
        pltpu.async_copy(ese.at[pl.ds(off, CH)], se[b], esem[b])
        pltpu.async_copy(edst.at[pl.ds(off, CH)], sd[b], esem[b])

    def _wait(b):
        pltpu.make_async_copy(ese.at[pl.ds(0, CH)], se[b], esem[b]).wait()
        pltpu.make_async_copy(edst.at[pl.ds(0, CH)], sd[b], esem[b]).wait()

    _start(0, 0)
    for ch in range(NCH):
        b = ch % 2
        if ch + 1 < NCH:
            _start(ch + 1, 1 - b)
        _wait(b)
        seb = se[b]
        sdb = sd[b]

        def _edge(i, icarry):
            s16 = pl.ds(i * 16, 16)
            e0 = seb[s16]
            d = sdb[s16]
            s = lax.bitwise_and(e0, 0xFFFF)
            t = lax.shift_right_logical(e0, 16)
            w = plsc.load_gather(wt, [t])
            xs = plsc.load_gather(x_local, [s])
            r = lax.shift_right_logical(d, 4)
            col = lax.bitwise_and(d, 15)
            plsc.addupdate_scatter(acc, [r, col], w * xs)
            return icarry

        lax.fori_loop(0, VPC, _edge, None)

    cvv = cv[...]
    bvv = bv[...]

    @pl.when(cid == 0)
    def _selfloop():
        def _sl(i, carry):
            r = sid * RPS + i
            acc[r, :] = acc[r, :] + cvv * x_local[pl.ds(r * 16, 16)] + bvv
            return carry

        lax.fori_loop(0, RPS, _sl, None)

    cri.wait()

    @pl.when(sid == 0)
    def _seed_part():
        pltpu.sync_copy(acc, part)

    plsc.subcore_barrier()

    @pl.when(sid != 0)
    def _scatter_add():
        cps = [
            pltpu.async_copy(acc.at[pl.ds(c * 112, 112)], part.at[ridx.at[c]],
                             rsem, add=True)
            for c in range(28)
        ]
        for cp in cps:
            cp.wait()

    plsc.subcore_barrier()

    pltpu.sync_copy(part.at[pl.ds(sid * RPS, RPS)], out.at[cid, sid])


def _fc_body(p0, p1, w0, w1, w2, w3, b, o):
    h = p0[...] + p1[...]
    for k, w in enumerate((w0, w1, w2, w3)):
        logits = lax.dot_general(h, w[0], (((1,), (1,)), ((), ())),
                                 preferred_element_type=jnp.float32)
        z = logits + b[0, :, pl.ds(k * 100, 100)]
        o[0, :, pl.ds(k * 100, 100)] = 1.0 / (1.0 + jnp.exp(-z))


_fc = pl.pallas_call(
    _fc_body,
    grid=(10,),
    in_specs=[
        pl.BlockSpec((8, 10000), lambda j: (0, 0)),
        pl.BlockSpec((8, 10000), lambda j: (0, 0)),
        pl.BlockSpec((1, 100, 10000), lambda j: (4 * j, 0, 0)),
        pl.BlockSpec((1, 100, 10000), lambda j: (4 * j + 1, 0, 0)),
        pl.BlockSpec((1, 100, 10000), lambda j: (4 * j + 2, 0, 0)),
        pl.BlockSpec((1, 100, 10000), lambda j: (4 * j + 3, 0, 0)),
        pl.BlockSpec((1, 1, 400), lambda j: (j, 0, 0)),
    ],
    out_specs=pl.BlockSpec((1, 8, 400), lambda j: (j, 0, 0)),
    out_shape=jax.ShapeDtypeStruct((10, 8, 400), jnp.float32),
)


def kernel(features, edge_index, etypes, W_rel1, W_loop1, b1,
           W_rel2, W_loop2, b2, W_rel3, W_loop3, b3, fc_w, fc_b):
    x0 = jnp.pad(features[:, 0], (0, NPAD - N))
    zero = jnp.zeros((NPAD,), jnp.float32)
    src = edge_index[0]
    dst = edge_index[1]
    ese = jnp.bitwise_or(src, jnp.left_shift(etypes, 16))

    def vec16(x):
        return jnp.full((16,), x.reshape(-1)[0], dtype=jnp.float32)

    def wpad(W):
        return jnp.pad(W[:, 0, 0], (0, 16 - W.shape[0]))

    wts = jnp.stack([wpad(W_rel1), wpad(W_rel2), wpad(W_rel3)])
    cvs = jnp.stack([vec16(W_loop1), vec16(W_loop2), vec16(W_loop3)])
    bvs = jnp.stack([vec16(b1), vec16(b2), vec16(b3)])
    ridx = jnp.arange(ROWS, dtype=jnp.int32).reshape(28, 112)

    def _layer_step(carry, wcb):
        pa, pb = carry
        w, c, b = wcb
        o = _rgc_layer(pa, pb, ese, dst, w, c, b, ridx).reshape(2, NPAD)
        return (o[0], o[1]), None

    (p0, p1), _ = lax.scan(_layer_step, (x0, zero), (wts, cvs, bvs))
    p = jnp.stack([p0, p1])

    h0 = jnp.pad(p[0, :N].reshape(5, 10000), ((0, 3), (0, 0)))
    h1 = jnp.pad(p[1, :N].reshape(5, 10000), ((0, 3), (0, 0)))
    w40 = fc_w.reshape(40, 100, 10000)
    out = _fc(h0, h1, w40, w40, w40, w40, fc_b.reshape(10, 1, 400))
    return out.transpose(1, 0, 2).reshape(8, 4000)[:5]


# speedup vs baseline: 1.6758x; 1.6758x over previous
import functools

import jax
import jax.numpy as jnp
from jax import lax
from jax.experimental import pallas as pl
from jax.experimental.pallas import tpu as pltpu
from jax.experimental.pallas import tpu_sc as plsc

N = 50000
NPAD = 50176
ROWS = 3136
RPS = 196
SL = 3136
E = 1600000
NW = 32
EPW = E // NW
CH = 2000
NCH = EPW // CH
VPC = CH // 16

_mesh = plsc.VectorSubcoreMesh(core_axis_name="c", subcore_axis_name="s")


@functools.partial(
    pl.kernel,
    out_type=jax.ShapeDtypeStruct((2, 16, RPS, 16), jnp.float32),
    mesh=_mesh,
    compiler_params=pltpu.CompilerParams(needs_layout_passes=False,
                                         use_tc_tiling_on_sc=False),
    scratch_types=[
        pltpu.VMEM((NPAD,), jnp.float32),
        pltpu.VMEM((ROWS, 16), jnp.float32),
        pltpu.VMEM((SL,), jnp.float32),
        pltpu.VMEM((SL,), jnp.float32),
        pltpu.VMEM((CH,), jnp.int32),
        pltpu.VMEM((CH,), jnp.int32),
        pltpu.VMEM((CH,), jnp.int32),
        pltpu.VMEM((CH,), jnp.int32),
        pltpu.VMEM((28, 112), jnp.int32),
        pltpu.VMEM((16,), jnp.float32),
        pltpu.VMEM((16,), jnp.float32),
        pltpu.VMEM((16,), jnp.float32),
        pltpu.VMEM_SHARED((NPAD,), jnp.float32),
        pltpu.VMEM_SHARED((ROWS, 16), jnp.float32),
        pltpu.SemaphoreType.DMA,
        pltpu.SemaphoreType.DMA,
        pltpu.SemaphoreType.DMA,
        pltpu.SemaphoreType.DMA,
        pltpu.SemaphoreType.DMA,
    ],
)
def _rgc_layer(pa, pb, ese, edst, wtab, cvec, bvec, ridx_in,
               out,
               x_local, acc, buf_a, buf_b, se0, se1, sd0, sd1,
               ridx, wt, cv, bv, x_shared, part,
               sem_a, sem_b, esem0, esem1, rsem):
    cid = lax.axis_index("c")
    sid = lax.axis_index("s")
    wid = cid * 16 + sid
    se = (se0, se1)
    sd = (sd0, sd1)
    esem = (esem0, esem1)

    pltpu.sync_copy(wtab, wt)
    pltpu.sync_copy(cvec, cv)
    pltpu.sync_copy(bvec, bv)

    cpa = pltpu.async_copy(pa.at[pl.ds(sid * SL, SL)], buf_a, sem_a)
    cpb = pltpu.async_copy(pb.at[pl.ds(sid * SL, SL)], buf_b, sem_b)
    cri = pltpu.async_copy(ridx_in, ridx, rsem)

    zeros = jnp.zeros((16,), jnp.float32)

    def _zero(r, carry):
        acc[r, :] = zeros
        return carry

    lax.fori_loop(0, ROWS, _zero, None)
    cpa.wait()
    cpb.wait()

    def _addp(i, carry):
        s16 = pl.ds(i * 16, 16)
        buf_a[s16] = buf_a[s16] + buf_b[s16]
        return carry

    lax.fori_loop(0, RPS, _addp, None)
    pltpu.sync_copy(buf_a, x_shared.at[pl.ds(sid * SL, SL)])
    plsc.subcore_barrier()

    pltpu.sync_copy(x_shared, x_local)

    ebase = wid * EPW

    def _start(ch, b):
        off = ebase + ch * CH
        pltpu.async_copy(ese.at[pl.ds(off, CH)], se[b], esem[b])
        pltpu.async_copy(edst.at[pl.ds(off, CH)], sd[b], esem[b])

    def _wait(b):
        pltpu.make_async_copy(ese.at[pl.ds(0, CH)], se[b], esem[b]).wait()
        pltpu.make_async_copy(edst.at[pl.ds(0, CH)], sd[b], esem[b]).wait()

    _start(0, 0)
    for ch in range(NCH):
        b = ch % 2
        if ch + 1 < NCH:
            _start(ch + 1, 1 - b)
        _wait(b)
        seb = se[b]
        sdb = sd[b]

        @plsc.parallel_loop(0, CH, step=16, unroll=4)
        def _edge(i):
            s16 = pl.ds(i, 16)
            e0 = seb[s16]
            d = sdb[s16]
            s = lax.bitwise_and(e0, 0xFFFF)
            t = lax.shift_right_logical(e0, 16)
            w = plsc.load_gather(wt, [t])
            xs = plsc.load_gather(x_local, [s])
            r = lax.shift_right_logical(d, 4)
            col = lax.bitwise_and(d, 15)
            plsc.addupdate_scatter(acc, [r, col], w * xs)

    cvv = cv[...]
    bvv = bv[...]

    @pl.when(cid == 0)
    def _selfloop():
        def _sl(i, carry):
            r = sid * RPS + i
            acc[r, :] = acc[r, :] + cvv * x_local[pl.ds(r * 16, 16)] + bvv
            return carry

        lax.fori_loop(0, RPS, _sl, None)

    cri.wait()

    @pl.when(sid == 0)
    def _seed_part():
        pltpu.sync_copy(acc, part)

    plsc.subcore_barrier()

    @pl.when(sid != 0)
    def _scatter_add():
        cps = [
            pltpu.async_copy(acc.at[pl.ds(c * 112, 112)], part.at[ridx.at[c]],
                             rsem, add=True)
            for c in range(28)
        ]
        for cp in cps:
            cp.wait()

    plsc.subcore_barrier()

    pltpu.sync_copy(part.at[pl.ds(sid * RPS, RPS)], out.at[cid, sid])


def _fc_body(p0, p1, w, b, o):
    h = p0[...] + p1[...]
    logits = lax.dot_general(h, w[0], (((1,), (1,)), ((), ())),
                             preferred_element_type=jnp.float32)
    z = logits + b[0]
    o[...] = (1.0 / (1.0 + jnp.exp(-z)))[None]


_fc = pl.pallas_call(
    _fc_body,
    grid=(10,),
    in_specs=[
        pl.BlockSpec((8, 10000), lambda j: (0, 0)),
        pl.BlockSpec((8, 10000), lambda j: (0, 0)),
        pl.BlockSpec((1, 400, 10000), lambda j: (j, 0, 0)),
        pl.BlockSpec((1, 1, 400), lambda j: (j, 0, 0)),
    ],
    out_specs=pl.BlockSpec((1, 8, 400), lambda j: (j, 0, 0)),
    out_shape=jax.ShapeDtypeStruct((10, 8, 400), jnp.float32),
)


def kernel(features, edge_index, etypes, W_rel1, W_loop1, b1,
           W_rel2, W_loop2, b2, W_rel3, W_loop3, b3, fc_w, fc_b):
    x0 = jnp.pad(features[:, 0], (0, NPAD - N))
    zero = jnp.zeros((NPAD,), jnp.float32)
    src = edge_index[0]
    dst = edge_index[1]
    ese = jnp.bitwise_or(src, jnp.left_shift(etypes, 16))

    def vec16(x):
        return jnp.full((16,), x.reshape(-1)[0], dtype=jnp.float32)

    def wpad(W):
        return jnp.pad(W[:, 0, 0], (0, 16 - W.shape[0]))

    wts = jnp.stack([wpad(W_rel1), wpad(W_rel2), wpad(W_rel3)])
    cvs = jnp.stack([vec16(W_loop1), vec16(W_loop2), vec16(W_loop3)])
    bvs = jnp.stack([vec16(b1), vec16(b2), vec16(b3)])
    ridx = jnp.arange(ROWS, dtype=jnp.int32).reshape(28, 112)

    def _layer_step(carry, wcb):
        pa, pb = carry
        w, c, b = wcb
        o = _rgc_layer(pa, pb, ese, dst, w, c, b, ridx).reshape(2, NPAD)
        return (o[0], o[1]), None

    (p0, p1), _ = lax.scan(_layer_step, (x0, zero), (wts, cvs, bvs))
    p = jnp.stack([p0, p1])

    h0 = jnp.pad(p[0, :N].reshape(5, 10000), ((0, 3), (0, 0)))
    h1 = jnp.pad(p[1, :N].reshape(5, 10000), ((0, 3), (0, 0)))
    out = _fc(h0, h1, fc_w.reshape(10, 400, 10000), fc_b.reshape(10, 1, 400))
    return out.transpose(1, 0, 2).reshape(8, 4000)[:5]


# speedup vs baseline: 1.9056x; 1.1371x over previous
import functools

import jax
import jax.numpy as jnp
from jax import lax
from jax.experimental import pallas as pl
from jax.experimental.pallas import tpu as pltpu
from jax.experimental.pallas import tpu_sc as plsc

N = 50000
NPAD = 50176
ROWS = 3136
RPS = 196
SL = 3136
E = 1600000
NW = 32
EPW = E // NW
CH = 2000
NCH = EPW // CH
VPC = CH // 16

_mesh = plsc.VectorSubcoreMesh(core_axis_name="c", subcore_axis_name="s")


@functools.partial(
    pl.kernel,
    out_type=jax.ShapeDtypeStruct((2, 16, RPS, 16), jnp.float32),
    mesh=_mesh,
    compiler_params=pltpu.CompilerParams(needs_layout_passes=False,
                                         use_tc_tiling_on_sc=False),
    scratch_types=[
        pltpu.VMEM((NPAD,), jnp.float32),
        pltpu.VMEM((ROWS, 16), jnp.float32),
        pltpu.VMEM((SL,), jnp.float32),
        pltpu.VMEM((SL,), jnp.float32),
        pltpu.VMEM((CH,), jnp.int32),
        pltpu.VMEM((CH,), jnp.int32),
        pltpu.VMEM((CH,), jnp.int32),
        pltpu.VMEM((CH,), jnp.int32),
        pltpu.VMEM((28, 112), jnp.int32),
        pltpu.VMEM((16,), jnp.float32),
        pltpu.VMEM((16,), jnp.float32),
        pltpu.VMEM((16,), jnp.float32),
        pltpu.VMEM_SHARED((NPAD,), jnp.float32),
        pltpu.VMEM_SHARED((ROWS, 16), jnp.float32),
        pltpu.SemaphoreType.DMA,
        pltpu.SemaphoreType.DMA,
        pltpu.SemaphoreType.DMA,
        pltpu.SemaphoreType.DMA,
        pltpu.SemaphoreType.DMA,
    ],
)
def _rgc_layer(pa, pb, ese, edst, wtab, cvec, bvec, ridx_in,
               out,
               x_local, acc, buf_a, buf_b, se0, se1, sd0, sd1,
               ridx, wt, cv, bv, x_shared, part,
               sem_a, sem_b, esem0, esem1, rsem):
    cid = lax.axis_index("c")
    sid = lax.axis_index("s")
    wid = cid * 16 + sid
    se = (se0, se1)
    sd = (sd0, sd1)
    esem = (esem0, esem1)

    pltpu.sync_copy(wtab, wt)
    pltpu.sync_copy(cvec, cv)
    pltpu.sync_copy(bvec, bv)

    cpa = pltpu.async_copy(pa.at[pl.ds(sid * SL, SL)], buf_a, sem_a)
    cpb = pltpu.async_copy(pb.at[pl.ds(sid * SL, SL)], buf_b, sem_b)
    cri = pltpu.async_copy(ridx_in, ridx, rsem)

    zeros = jnp.zeros((16,), jnp.float32)

    @plsc.parallel_loop(0, ROWS, step=1, unroll=8)
    def _zero(r):
        acc[r, :] = zeros

    cpa.wait()
    cpb.wait()

    @plsc.parallel_loop(0, SL, step=16, unroll=4)
    def _addp(i):
        s16 = pl.ds(i, 16)
        buf_a[s16] = buf_a[s16] + buf_b[s16]
    pltpu.sync_copy(buf_a, x_shared.at[pl.ds(sid * SL, SL)])
    plsc.subcore_barrier()

    pltpu.sync_copy(x_shared, x_local)

    ebase = wid * EPW

    def _start(ch, b):
        off = ebase + ch * CH
        pltpu.async_copy(ese.at[pl.ds(off, CH)], se[b], esem[b])
        pltpu.async_copy(edst.at[pl.ds(off, CH)], sd[b], esem[b])

    def _wait(b):
        pltpu.make_async_copy(ese.at[pl.ds(0, CH)], se[b], esem[b]).wait()
        pltpu.make_async_copy(edst.at[pl.ds(0, CH)], sd[b], esem[b]).wait()

    _start(0, 0)
    for ch in range(NCH):
        b = ch % 2
        if ch + 1 < NCH:
            _start(ch + 1, 1 - b)
        _wait(b)
        seb = se[b]
        sdb = sd[b]

        @plsc.parallel_loop(0, CH, step=16, unroll=8)
        def _edge(i):
            s16 = pl.ds(i, 16)
            e0 = seb[s16]
            d = sdb[s16]
            s = lax.bitwise_and(e0, 0xFFFF)
            t = lax.shift_right_logical(e0, 16)
            w = plsc.load_gather(wt, [t])
            xs = plsc.load_gather(x_local, [s])
            r = lax.shift_right_logical(d, 4)
            col = lax.bitwise_and(d, 15)
            plsc.addupdate_scatter(acc, [r, col], w * xs)

    cvv = cv[...]
    bvv = bv[...]

    @pl.when(cid == 0)
    def _selfloop():
        @plsc.parallel_loop(0, RPS, step=1, unroll=4)
        def _sl(i):
            r = sid * RPS + i
            acc[r, :] = acc[r, :] + cvv * x_local[pl.ds(r * 16, 16)] + bvv

    cri.wait()

    @pl.when(sid == 0)
    def _seed_part():
        pltpu.sync_copy(acc, part)

    plsc.subcore_barrier()

    @pl.when(sid != 0)
    def _scatter_add():
        cps = [
            pltpu.async_copy(acc.at[pl.ds(c * 112, 112)], part.at[ridx.at[c]],
                             rsem, add=True)
            for c in range(28)
        ]
        for cp in cps:
            cp.wait()

    plsc.subcore_barrier()

    pltpu.sync_copy(part.at[pl.ds(sid * RPS, RPS)], out.at[cid, sid])


def _fc_body(p0, p1, w, b, o):
    h = p0[...] + p1[...]
    logits = lax.dot_general(h, w[0], (((1,), (1,)), ((), ())),
                             preferred_element_type=jnp.float32)
    z = logits + b[0]
    o[...] = (1.0 / (1.0 + jnp.exp(-z)))[None]


_fc = pl.pallas_call(
    _fc_body,
    grid=(10,),
    in_specs=[
        pl.BlockSpec((8, 10000), lambda j: (0, 0)),
        pl.BlockSpec((8, 10000), lambda j: (0, 0)),
        pl.BlockSpec((1, 400, 10000), lambda j: (j, 0, 0)),
        pl.BlockSpec((1, 1, 400), lambda j: (j, 0, 0)),
    ],
    out_specs=pl.BlockSpec((1, 8, 400), lambda j: (j, 0, 0)),
    out_shape=jax.ShapeDtypeStruct((10, 8, 400), jnp.float32),
)


def kernel(features, edge_index, etypes, W_rel1, W_loop1, b1,
           W_rel2, W_loop2, b2, W_rel3, W_loop3, b3, fc_w, fc_b):
    x0 = jnp.pad(features[:, 0], (0, NPAD - N))
    zero = jnp.zeros((NPAD,), jnp.float32)
    src = edge_index[0]
    dst = edge_index[1]
    ese = jnp.bitwise_or(src, jnp.left_shift(etypes, 16))

    def vec16(x):
        return jnp.full((16,), x.reshape(-1)[0], dtype=jnp.float32)

    def wpad(W):
        return jnp.pad(W[:, 0, 0], (0, 16 - W.shape[0]))

    wts = jnp.stack([wpad(W_rel1), wpad(W_rel2), wpad(W_rel3)])
    cvs = jnp.stack([vec16(W_loop1), vec16(W_loop2), vec16(W_loop3)])
    bvs = jnp.stack([vec16(b1), vec16(b2), vec16(b3)])
    ridx = jnp.arange(ROWS, dtype=jnp.int32).reshape(28, 112)

    def _layer_step(carry, wcb):
        pa, pb = carry
        w, c, b = wcb
        o = _rgc_layer(pa, pb, ese, dst, w, c, b, ridx).reshape(2, NPAD)
        return (o[0], o[1]), None

    (p0, p1), _ = lax.scan(_layer_step, (x0, zero), (wts, cvs, bvs))
    p = jnp.stack([p0, p1])

    h0 = jnp.pad(p[0, :N].reshape(5, 10000), ((0, 3), (0, 0)))
    h1 = jnp.pad(p[1, :N].reshape(5, 10000), ((0, 3), (0, 0)))
    out = _fc(h0, h1, fc_w.reshape(10, 400, 10000), fc_b.reshape(10, 1, 400))
    return out.transpose(1, 0, 2).reshape(8, 4000)[:5]
